# Initial kernel scaffold; baseline (speedup 1.0000x reference)
import jax
import jax.numpy as jnp
from jax.experimental import pallas as pl


def kernel(z, pos, batch, params):
    raise NotImplementedError("write your pallas kernel here")


# speedup vs baseline: 4.5491x; 4.5491x over previous
import numpy as np
import jax
import jax.numpy as jnp
from jax.experimental import pallas as pl
from jax.experimental.pallas import tpu as pltpu

N = 10000
NG = 512
HID = 64
NRBF = 32
NLAYERS = 6
NHEADS = 8
DH = HID // NHEADS
K = 16
CUTOFF = 5.0
MAXZ = 100
NSPH = 8
E = N * K

G = 10
RB = 125

_ALPHA = 5.0 / CUTOFF
_MEANS = np.linspace(float(np.exp(-CUTOFF)), 1.0, NRBF)
_BETA = float((2.0 / NRBF * (1.0 - np.exp(-CUTOFF))) ** -2)
_S3 = float(np.sqrt(3.0))


def _silu(x):
    return x * jax.nn.sigmoid(x)


def _layernorm(x, w, b):
    m = jnp.mean(x, axis=-1, keepdims=True)
    v = jnp.mean((x - m) ** 2, axis=-1, keepdims=True)
    return (x - m) / jnp.sqrt(v + 1e-5) * w + b


def _build_graph(pos):
    norms = jnp.sum(pos * pos, axis=1)
    idx_list, valid_list = [], []
    B = 1000
    for s in range(0, N, B):
        blk = pos[s:s + B]
        d2 = norms[s:s + B, None] + norms[None, :] - 2.0 * (blk @ pos.T)
        rows = jnp.arange(B)
        d2 = d2.at[rows, rows + s].set(jnp.inf)
        d2 = jnp.where(d2 <= CUTOFF * CUTOFF, d2, jnp.inf)
        neg, idx = jax.lax.top_k(-d2, K)
        idx_list.append(idx)
        valid_list.append(jnp.isfinite(neg))
    nbr = jnp.concatenate(idx_list, axis=0)
    valid = jnp.concatenate(valid_list, axis=0)
    return nbr.reshape(-1), valid.reshape(-1)


def _edge_feat_body(rv_ref, mask_ref, cw_ref, rbf_ref, sph_ref):
    x = rv_ref[0, 0]
    y = rv_ref[1, 0]
    z = rv_ref[2, 0]
    d = jnp.sqrt(x * x + y * y + z * z + 1e-12)
    inv = 1.0 / d
    ux, uy, uz = x * inv, y * inv, z * inv
    cw = 0.5 * (jnp.cos(np.pi * jnp.minimum(d, CUTOFF) / CUTOFF) + 1.0)
    cw = jnp.where((mask_ref[0] > 0.5) & (d < CUTOFF), cw, 0.0)
    cw_ref[0] = cw
    ed = jnp.exp(-_ALPHA * d)
    for j in range(NRBF):
        rbf_ref[j, 0] = cw * jnp.exp(-_BETA * (ed - _MEANS[j]) ** 2)
    sph_ref[0, 0] = cw * ux
    sph_ref[1, 0] = cw * uy
    sph_ref[2, 0] = cw * uz
    sph_ref[3, 0] = cw * (_S3 * ux * uy)
    sph_ref[4, 0] = cw * (_S3 * uy * uz)
    sph_ref[5, 0] = cw * (0.5 * (3.0 * uz * uz - 1.0))
    sph_ref[6, 0] = cw * (_S3 * ux * uz)
    sph_ref[7, 0] = cw * (0.5 * _S3 * (ux * ux - uy * uy))


def _edge_features(rvec, maskf):
    rv3 = rvec.T.reshape(3, G, RB, 128)
    m2 = maskf.reshape(G, RB, 128)
    cw, rbf, sph = pl.pallas_call(
        _edge_feat_body,
        grid=(G,),
        in_specs=[
            pl.BlockSpec((3, 1, RB, 128), lambda i: (0, i, 0, 0)),
            pl.BlockSpec((1, RB, 128), lambda i: (i, 0, 0)),
        ],
        out_specs=[
            pl.BlockSpec((1, RB, 128), lambda i: (i, 0, 0)),
            pl.BlockSpec((NRBF, 1, RB, 128), lambda i: (0, i, 0, 0)),
            pl.BlockSpec((NSPH, 1, RB, 128), lambda i: (0, i, 0, 0)),
        ],
        out_shape=[
            jax.ShapeDtypeStruct((G, RB, 128), jnp.float32),
            jax.ShapeDtypeStruct((NRBF, G, RB, 128), jnp.float32),
            jax.ShapeDtypeStruct((NSPH, G, RB, 128), jnp.float32),
        ],
    )(rv3, m2)
    cw = cw.reshape(E)
    rbf = rbf.reshape(NRBF, E).T
    sph = sph.reshape(NSPH, E).T
    return cw, rbf, sph


def _layer_step(x, vec, lp, src, cw_nk, rbf, sph_nk):
    h = _layernorm(x, lp['ln_w'], lp['ln_b'])
    q = (h @ lp['Wq']).reshape(N, NHEADS, DH)
    k = (h @ lp['Wk']).reshape(N, HID)
    v = (h @ lp['Wv']).reshape(N, HID)
    dk = _silu(rbf @ lp['Wdk']).reshape(N, K, NHEADS, DH)
    dv = _silu(rbf @ lp['Wdv']).reshape(N, K, NHEADS, 3 * DH)
    dv_x, dv_v1, dv_v2 = jnp.split(dv, 3, axis=-1)
    vecp = jnp.einsum('nsh,hk->nsk', vec, lp['Wvec'])
    vec1, vec2, vec3 = jnp.split(vecp, 3, axis=-1)
    vec_dot = jnp.sum(vec1 * vec2, axis=1)
    ksrc = k[src].reshape(N, K, NHEADS, DH)
    vsrc = v[src].reshape(N, K, NHEADS, DH)
    attn = jnp.sum(q[:, None] * ksrc * dk, axis=-1) / np.sqrt(DH)
    attn = _silu(attn) * cw_nk[..., None]
    xm = vsrc * dv_x * attn[..., None]
    s1 = (vsrc * dv_v1 * attn[..., None]).reshape(N, K, HID)
    s2 = (vsrc * dv_v2 * attn[..., None]).reshape(N, K, HID)
    x_agg = xm.reshape(N, K, HID).sum(axis=1)
    vec_src = vec[src].reshape(N, K, NSPH, HID)
    vec_agg = (jnp.sum(vec_src * s1[:, :, None, :], axis=1)
               + jnp.einsum('nks,nkh->nsh', sph_nk, s2))
    o = x_agg @ lp['Wo']
    o1, o2, o3 = jnp.split(o, 3, axis=-1)
    x = x + o2 + o3 * vec_dot
    vec = vec + vec_agg + vec3 * o1[:, None, :]
    return x, vec


def kernel(z, pos, batch, params):
    src, mask = _build_graph(pos)
    dst = jnp.repeat(jnp.arange(N), K)
    rvec = pos[src] - pos[dst]
    cw, rbf, sph = _edge_features(rvec, mask.astype(jnp.float32))
    cw_nk = cw.reshape(N, K)
    sph_nk = sph.reshape(N, K, NSPH)
    x = params['embed'][z]
    vec = jnp.zeros((N, NSPH, HID), jnp.float32)
    lkeys = ['Wq', 'Wk', 'Wv', 'Wdk', 'Wdv', 'Wvec', 'Wo', 'ln_w', 'ln_b']
    for i in range(NLAYERS):
        lp = {kk: params[kk][i] for kk in lkeys}
        x, vec = _layer_step(x, vec, lp, src, cw_nk, rbf, sph_nk)
    h = _silu(x @ params['Wout1'] + params['bout1'])
    atom_energy = h @ params['Wout2'] + params['bout2']
    out = jax.ops.segment_sum(atom_energy, batch, num_segments=NG)
    return out


# speedup vs baseline: 4.7912x; 1.0532x over previous
import functools

import numpy as np
import jax
import jax.numpy as jnp
from jax import lax
from jax.experimental import pallas as pl
from jax.experimental.pallas import tpu as pltpu
from jax.experimental.pallas import tpu_sc as plsc

N = 10000
NG = 512
HID = 64
NRBF = 32
NLAYERS = 6
NHEADS = 8
DH = HID // NHEADS
K = 16
CUTOFF = 5.0
MAXZ = 100
NSPH = 8
E = N * K

G = 10
RB = 125

_ALPHA = 5.0 / CUTOFF
_MEANS = np.linspace(float(np.exp(-CUTOFF)), 1.0, NRBF)
_BETA = float((2.0 / NRBF * (1.0 - np.exp(-CUTOFF))) ** -2)
_S3 = float(np.sqrt(3.0))


def _silu(x):
    return x * jax.nn.sigmoid(x)


def _layernorm(x, w, b):
    m = jnp.mean(x, axis=-1, keepdims=True)
    v = jnp.mean((x - m) ** 2, axis=-1, keepdims=True)
    return (x - m) / jnp.sqrt(v + 1e-5) * w + b


def _build_graph(pos):
    norms = jnp.sum(pos * pos, axis=1)
    idx_list, valid_list = [], []
    B = 1000
    for s in range(0, N, B):
        blk = pos[s:s + B]
        d2 = norms[s:s + B, None] + norms[None, :] - 2.0 * (blk @ pos.T)
        rows = jnp.arange(B)
        d2 = d2.at[rows, rows + s].set(jnp.inf)
        d2 = jnp.where(d2 <= CUTOFF * CUTOFF, d2, jnp.inf)
        neg, idx = jax.lax.top_k(-d2, K)
        idx_list.append(idx)
        valid_list.append(jnp.isfinite(neg))
    nbr = jnp.concatenate(idx_list, axis=0)
    valid = jnp.concatenate(valid_list, axis=0)
    return nbr.reshape(-1), valid.reshape(-1)


def _edge_feat_body(rv_ref, mask_ref, cw_ref, rbf_ref, sph_ref):
    x = rv_ref[0, 0]
    y = rv_ref[1, 0]
    z = rv_ref[2, 0]
    d = jnp.sqrt(x * x + y * y + z * z + 1e-12)
    inv = 1.0 / d
    ux, uy, uz = x * inv, y * inv, z * inv
    cw = 0.5 * (jnp.cos(np.pi * jnp.minimum(d, CUTOFF) / CUTOFF) + 1.0)
    cw = jnp.where((mask_ref[0] > 0.5) & (d < CUTOFF), cw, 0.0)
    cw_ref[0] = cw
    ed = jnp.exp(-_ALPHA * d)
    for j in range(NRBF):
        rbf_ref[j, 0] = cw * jnp.exp(-_BETA * (ed - _MEANS[j]) ** 2)
    sph_ref[0, 0] = cw * ux
    sph_ref[1, 0] = cw * uy
    sph_ref[2, 0] = cw * uz
    sph_ref[3, 0] = cw * (_S3 * ux * uy)
    sph_ref[4, 0] = cw * (_S3 * uy * uz)
    sph_ref[5, 0] = cw * (0.5 * (3.0 * uz * uz - 1.0))
    sph_ref[6, 0] = cw * (_S3 * ux * uz)
    sph_ref[7, 0] = cw * (0.5 * _S3 * (ux * ux - uy * uy))


def _edge_features(rvec, maskf):
    rv3 = rvec.T.reshape(3, G, RB, 128)
    m2 = maskf.reshape(G, RB, 128)
    cw, rbf, sph = pl.pallas_call(
        _edge_feat_body,
        grid=(G,),
        in_specs=[
            pl.BlockSpec((3, 1, RB, 128), lambda i: (0, i, 0, 0)),
            pl.BlockSpec((1, RB, 128), lambda i: (i, 0, 0)),
        ],
        out_specs=[
            pl.BlockSpec((1, RB, 128), lambda i: (i, 0, 0)),
            pl.BlockSpec((NRBF, 1, RB, 128), lambda i: (0, i, 0, 0)),
            pl.BlockSpec((NSPH, 1, RB, 128), lambda i: (0, i, 0, 0)),
        ],
        out_shape=[
            jax.ShapeDtypeStruct((G, RB, 128), jnp.float32),
            jax.ShapeDtypeStruct((NRBF, G, RB, 128), jnp.float32),
            jax.ShapeDtypeStruct((NSPH, G, RB, 128), jnp.float32),
        ],
    )(rv3, m2)
    cw = cw.reshape(E)
    rbf = rbf.reshape(NRBF, E).T
    sph = sph.reshape(NSPH, E).T
    return cw, rbf, sph


NW = 32
NPAD = 10240
PNW = NPAD // NW
VD = NSPH * HID


def _vec_agg_body(vec_hbm, s1_hbm, idx_hbm, out_hbm, idx_v, gbuf, s1buf,
                  ostage, gsem, ssem, osem):
    wid = lax.axis_index("s") * 2 + lax.axis_index("c")
    base = wid * PNW
    pltpu.sync_copy(idx_hbm.at[pl.ds(base, PNW)], idx_v)

    def node(i, _):
        pltpu.async_copy(vec_hbm.at[idx_v[i]], gbuf, gsem).wait()
        pltpu.async_copy(s1_hbm.at[pl.ds(base + i, 1)], s1buf, ssem).wait()
        for s in range(NSPH):
            for j in range(HID // 16):
                acc = jnp.zeros((16,), jnp.float32)
                for k in range(K):
                    acc = acc + (gbuf[k, pl.ds(s * HID + j * 16, 16)]
                                 * s1buf[0, pl.ds(k * HID + j * 16, 16)])
                ostage[0, pl.ds(s * HID + j * 16, 16)] = acc
        pltpu.async_copy(ostage, out_hbm.at[pl.ds(base + i, 1)], osem).wait()
        return 0

    lax.fori_loop(0, PNW, node, 0)


@functools.partial(
    pl.kernel,
    mesh=plsc.VectorSubcoreMesh(core_axis_name="c", subcore_axis_name="s"),
    out_type=jax.ShapeDtypeStruct((NPAD, VD), jnp.float32),
    scratch_types=[
        pltpu.VMEM((PNW, K), jnp.int32),
        pltpu.VMEM((K, VD), jnp.float32),
        pltpu.VMEM((1, K * HID), jnp.float32),
        pltpu.VMEM((1, VD), jnp.float32),
        pltpu.SemaphoreType.DMA,
        pltpu.SemaphoreType.DMA,
        pltpu.SemaphoreType.DMA,
    ],
)
def _vec_agg_sc(vec_hbm, s1_hbm, idx_hbm, out_hbm, *scratch):
    _vec_agg_body(vec_hbm, s1_hbm, idx_hbm, out_hbm, *scratch)


def _vec_agg(vec, s1, src_nk):
    vec_tbl = jnp.pad(vec.reshape(N, VD), ((0, NPAD - N), (0, 0)))
    s1_tbl = jnp.pad(s1.reshape(N, K * HID), ((0, NPAD - N), (0, 0)))
    idx = jnp.pad(src_nk, ((0, NPAD - N), (0, 0)))
    out = _vec_agg_sc(vec_tbl, s1_tbl, idx)
    return out[:N].reshape(N, NSPH, HID)


def _layer_step(x, vec, lp, src, src_nk, cw_nk, rbf, sph_nk):
    h = _layernorm(x, lp['ln_w'], lp['ln_b'])
    q = (h @ lp['Wq']).reshape(N, NHEADS, DH)
    k = (h @ lp['Wk']).reshape(N, HID)
    v = (h @ lp['Wv']).reshape(N, HID)
    dk = _silu(rbf @ lp['Wdk']).reshape(N, K, NHEADS, DH)
    dv = _silu(rbf @ lp['Wdv']).reshape(N, K, NHEADS, 3 * DH)
    dv_x, dv_v1, dv_v2 = jnp.split(dv, 3, axis=-1)
    vecp = jnp.einsum('nsh,hk->nsk', vec, lp['Wvec'])
    vec1, vec2, vec3 = jnp.split(vecp, 3, axis=-1)
    vec_dot = jnp.sum(vec1 * vec2, axis=1)
    ksrc = k[src].reshape(N, K, NHEADS, DH)
    vsrc = v[src].reshape(N, K, NHEADS, DH)
    attn = jnp.sum(q[:, None] * ksrc * dk, axis=-1) / np.sqrt(DH)
    attn = _silu(attn) * cw_nk[..., None]
    xm = vsrc * dv_x * attn[..., None]
    s1 = (vsrc * dv_v1 * attn[..., None]).reshape(N, K, HID)
    s2 = (vsrc * dv_v2 * attn[..., None]).reshape(N, K, HID)
    x_agg = xm.reshape(N, K, HID).sum(axis=1)
    vec_agg = _vec_agg(vec, s1, src_nk) + jnp.einsum('nks,nkh->nsh', sph_nk, s2)
    o = x_agg @ lp['Wo']
    o1, o2, o3 = jnp.split(o, 3, axis=-1)
    x = x + o2 + o3 * vec_dot
    vec = vec + vec_agg + vec3 * o1[:, None, :]
    return x, vec


def kernel(z, pos, batch, params):
    src, mask = _build_graph(pos)
    dst = jnp.repeat(jnp.arange(N), K)
    rvec = pos[src] - pos[dst]
    cw, rbf, sph = _edge_features(rvec, mask.astype(jnp.float32))
    cw_nk = cw.reshape(N, K)
    sph_nk = sph.reshape(N, K, NSPH)
    x = params['embed'][z]
    vec = jnp.zeros((N, NSPH, HID), jnp.float32)
    lkeys = ['Wq', 'Wk', 'Wv', 'Wdk', 'Wdv', 'Wvec', 'Wo', 'ln_w', 'ln_b']
    src_nk = src.reshape(N, K).astype(jnp.int32)
    for i in range(NLAYERS):
        lp = {kk: params[kk][i] for kk in lkeys}
        x, vec = _layer_step(x, vec, lp, src, src_nk, cw_nk, rbf, sph_nk)
    h = _silu(x @ params['Wout1'] + params['bout1'])
    atom_energy = h @ params['Wout2'] + params['bout2']
    out = jax.ops.segment_sum(atom_energy, batch, num_segments=NG)
    return out


# speedup vs baseline: 4.9761x; 1.0386x over previous
import functools

import numpy as np
import jax
import jax.numpy as jnp
from jax import lax
from jax.experimental import pallas as pl
from jax.experimental.pallas import tpu as pltpu
from jax.experimental.pallas import tpu_sc as plsc

N = 10000
NG = 512
HID = 64
NRBF = 32
NLAYERS = 6
NHEADS = 8
DH = HID // NHEADS
K = 16
CUTOFF = 5.0
MAXZ = 100
NSPH = 8
E = N * K

G = 10
RB = 125

_ALPHA = 5.0 / CUTOFF
_MEANS = np.linspace(float(np.exp(-CUTOFF)), 1.0, NRBF)
_BETA = float((2.0 / NRBF * (1.0 - np.exp(-CUTOFF))) ** -2)
_S3 = float(np.sqrt(3.0))


def _silu(x):
    return x * jax.nn.sigmoid(x)


def _layernorm(x, w, b):
    m = jnp.mean(x, axis=-1, keepdims=True)
    v = jnp.mean((x - m) ** 2, axis=-1, keepdims=True)
    return (x - m) / jnp.sqrt(v + 1e-5) * w + b


def _build_graph(pos):
    norms = jnp.sum(pos * pos, axis=1)
    idx_list, valid_list = [], []
    B = 1000
    for s in range(0, N, B):
        blk = pos[s:s + B]
        d2 = norms[s:s + B, None] + norms[None, :] - 2.0 * (blk @ pos.T)
        rows = jnp.arange(B)
        d2 = d2.at[rows, rows + s].set(jnp.inf)
        d2 = jnp.where(d2 <= CUTOFF * CUTOFF, d2, jnp.inf)
        neg, idx = jax.lax.top_k(-d2, K)
        idx_list.append(idx)
        valid_list.append(jnp.isfinite(neg))
    nbr = jnp.concatenate(idx_list, axis=0)
    valid = jnp.concatenate(valid_list, axis=0)
    return nbr.reshape(-1), valid.reshape(-1)


def _edge_feat_body(rv_ref, mask_ref, cw_ref, rbf_ref, sph_ref):
    x = rv_ref[0, 0]
    y = rv_ref[1, 0]
    z = rv_ref[2, 0]
    d = jnp.sqrt(x * x + y * y + z * z + 1e-12)
    inv = 1.0 / d
    ux, uy, uz = x * inv, y * inv, z * inv
    cw = 0.5 * (jnp.cos(np.pi * jnp.minimum(d, CUTOFF) / CUTOFF) + 1.0)
    cw = jnp.where((mask_ref[0] > 0.5) & (d < CUTOFF), cw, 0.0)
    cw_ref[0] = cw
    ed = jnp.exp(-_ALPHA * d)
    for j in range(NRBF):
        rbf_ref[j, 0] = cw * jnp.exp(-_BETA * (ed - _MEANS[j]) ** 2)
    sph_ref[0, 0] = cw * ux
    sph_ref[1, 0] = cw * uy
    sph_ref[2, 0] = cw * uz
    sph_ref[3, 0] = cw * (_S3 * ux * uy)
    sph_ref[4, 0] = cw * (_S3 * uy * uz)
    sph_ref[5, 0] = cw * (0.5 * (3.0 * uz * uz - 1.0))
    sph_ref[6, 0] = cw * (_S3 * ux * uz)
    sph_ref[7, 0] = cw * (0.5 * _S3 * (ux * ux - uy * uy))


def _edge_features(rvec, maskf):
    rv3 = rvec.T.reshape(3, G, RB, 128)
    m2 = maskf.reshape(G, RB, 128)
    cw, rbf, sph = pl.pallas_call(
        _edge_feat_body,
        grid=(G,),
        in_specs=[
            pl.BlockSpec((3, 1, RB, 128), lambda i: (0, i, 0, 0)),
            pl.BlockSpec((1, RB, 128), lambda i: (i, 0, 0)),
        ],
        out_specs=[
            pl.BlockSpec((1, RB, 128), lambda i: (i, 0, 0)),
            pl.BlockSpec((NRBF, 1, RB, 128), lambda i: (0, i, 0, 0)),
            pl.BlockSpec((NSPH, 1, RB, 128), lambda i: (0, i, 0, 0)),
        ],
        out_shape=[
            jax.ShapeDtypeStruct((G, RB, 128), jnp.float32),
            jax.ShapeDtypeStruct((NRBF, G, RB, 128), jnp.float32),
            jax.ShapeDtypeStruct((NSPH, G, RB, 128), jnp.float32),
        ],
    )(rv3, m2)
    cw = cw.reshape(E)
    rbf = rbf.reshape(NRBF, E).T
    sph = sph.reshape(NSPH, E).T
    return cw, rbf, sph


NW = 32
NPAD = 10240
PNW = NPAD // NW
VD = NSPH * HID


NB = 8


def _vec_agg_body(vec_hbm, s1_hbm, idx_hbm, out_hbm, idx_v, gbuf, s1buf,
                  ostage, gsem, ssem, osem):
    wid = lax.axis_index("s") * 2 + lax.axis_index("c")
    base = wid * PNW
    pltpu.sync_copy(idx_hbm.at[pl.ds(base, PNW)], idx_v)

    def issue(b, i):
        pltpu.make_async_copy(vec_hbm.at[idx_v[i]], gbuf.at[b],
                              gsem.at[b]).start()
        pltpu.make_async_copy(s1_hbm.at[pl.ds(base + i, 1)], s1buf.at[b],
                              ssem.at[b]).start()

    for b in range(NB):
        issue(b, b)

    def compute(b, i):
        pltpu.make_async_copy(vec_hbm.at[pl.ds(0, K)], gbuf.at[b],
                              gsem.at[b]).wait()
        pltpu.make_async_copy(s1_hbm.at[pl.ds(0, 1)], s1buf.at[b],
                              ssem.at[b]).wait()

        def col(c, _):
            s = c // 4
            j = c - s * 4
            off = s * HID + j * 16
            acc = jnp.zeros((16,), jnp.float32)
            for k in range(K):
                acc = acc + (gbuf[b, k, pl.ds(off, 16)]
                             * s1buf[b, 0, pl.ds(k * HID + j * 16, 16)])
            ostage[b, 0, pl.ds(off, 16)] = acc
            return 0

        lax.fori_loop(0, NSPH * 4, col, 0)
        pltpu.make_async_copy(ostage.at[b], out_hbm.at[pl.ds(base + i, 1)],
                              osem.at[b]).start()

    def outer(g, _):
        for b in range(NB):
            i = g * NB + b

            @pl.when(g > 0)
            def _():
                pltpu.make_async_copy(
                    ostage.at[b], out_hbm.at[pl.ds(base, 1)],
                    osem.at[b]).wait()

            compute(b, i)

            @pl.when(g < PNW // NB - 1)
            def _():
                issue(b, i + NB)

        return 0

    lax.fori_loop(0, PNW // NB, outer, 0)
    for b in range(NB):
        pltpu.make_async_copy(ostage.at[b], out_hbm.at[pl.ds(base, 1)],
                              osem.at[b]).wait()


@functools.partial(
    pl.kernel,
    mesh=plsc.VectorSubcoreMesh(core_axis_name="c", subcore_axis_name="s"),
    out_type=jax.ShapeDtypeStruct((NPAD, VD), jnp.float32),
    scratch_types=[
        pltpu.VMEM((PNW, K), jnp.int32),
        pltpu.VMEM((NB, K, VD), jnp.float32),
        pltpu.VMEM((NB, 1, K * HID), jnp.float32),
        pltpu.VMEM((NB, 1, VD), jnp.float32),
        pltpu.SemaphoreType.DMA((NB,)),
        pltpu.SemaphoreType.DMA((NB,)),
        pltpu.SemaphoreType.DMA((NB,)),
    ],
)
def _vec_agg_sc(vec_hbm, s1_hbm, idx_hbm, out_hbm, *scratch):
    _vec_agg_body(vec_hbm, s1_hbm, idx_hbm, out_hbm, *scratch)


def _vec_agg(vec, s1, src_nk):
    vec_tbl = jnp.pad(vec.reshape(N, VD), ((0, NPAD - N), (0, 0)))
    s1_tbl = jnp.pad(s1.reshape(N, K * HID), ((0, NPAD - N), (0, 0)))
    idx = jnp.pad(src_nk, ((0, NPAD - N), (0, 0)))
    out = _vec_agg_sc(vec_tbl, s1_tbl, idx)
    return out[:N].reshape(N, NSPH, HID)


def _layer_step(x, vec, lp, src, src_nk, cw_nk, rbf, sph_nk):
    h = _layernorm(x, lp['ln_w'], lp['ln_b'])
    q = (h @ lp['Wq']).reshape(N, NHEADS, DH)
    k = (h @ lp['Wk']).reshape(N, HID)
    v = (h @ lp['Wv']).reshape(N, HID)
    dk = _silu(rbf @ lp['Wdk']).reshape(N, K, NHEADS, DH)
    dv = _silu(rbf @ lp['Wdv']).reshape(N, K, NHEADS, 3 * DH)
    dv_x, dv_v1, dv_v2 = jnp.split(dv, 3, axis=-1)
    vecp = jnp.einsum('nsh,hk->nsk', vec, lp['Wvec'])
    vec1, vec2, vec3 = jnp.split(vecp, 3, axis=-1)
    vec_dot = jnp.sum(vec1 * vec2, axis=1)
    ksrc = k[src].reshape(N, K, NHEADS, DH)
    vsrc = v[src].reshape(N, K, NHEADS, DH)
    attn = jnp.sum(q[:, None] * ksrc * dk, axis=-1) / np.sqrt(DH)
    attn = _silu(attn) * cw_nk[..., None]
    xm = vsrc * dv_x * attn[..., None]
    s1 = (vsrc * dv_v1 * attn[..., None]).reshape(N, K, HID)
    s2 = (vsrc * dv_v2 * attn[..., None]).reshape(N, K, HID)
    x_agg = xm.reshape(N, K, HID).sum(axis=1)
    vec_agg = _vec_agg(vec, s1, src_nk) + jnp.einsum('nks,nkh->nsh', sph_nk, s2)
    o = x_agg @ lp['Wo']
    o1, o2, o3 = jnp.split(o, 3, axis=-1)
    x = x + o2 + o3 * vec_dot
    vec = vec + vec_agg + vec3 * o1[:, None, :]
    return x, vec


def kernel(z, pos, batch, params):
    src, mask = _build_graph(pos)
    dst = jnp.repeat(jnp.arange(N), K)
    rvec = pos[src] - pos[dst]
    cw, rbf, sph = _edge_features(rvec, mask.astype(jnp.float32))
    cw_nk = cw.reshape(N, K)
    sph_nk = sph.reshape(N, K, NSPH)
    x = params['embed'][z]
    vec = jnp.zeros((N, NSPH, HID), jnp.float32)
    lkeys = ['Wq', 'Wk', 'Wv', 'Wdk', 'Wdv', 'Wvec', 'Wo', 'ln_w', 'ln_b']
    src_nk = src.reshape(N, K).astype(jnp.int32)
    for i in range(NLAYERS):
        lp = {kk: params[kk][i] for kk in lkeys}
        x, vec = _layer_step(x, vec, lp, src, src_nk, cw_nk, rbf, sph_nk)
    h = _silu(x @ params['Wout1'] + params['bout1'])
    atom_energy = h @ params['Wout2'] + params['bout2']
    out = jax.ops.segment_sum(atom_energy, batch, num_segments=NG)
    return out


# speedup vs baseline: 9.1183x; 1.8324x over previous
import functools

import numpy as np
import jax
import jax.numpy as jnp
from jax import lax
from jax.experimental import pallas as pl
from jax.experimental.pallas import tpu as pltpu
from jax.experimental.pallas import tpu_sc as plsc

N = 10000
NG = 512
HID = 64
NRBF = 32
NLAYERS = 6
NHEADS = 8
DH = HID // NHEADS
K = 16
CUTOFF = 5.0
MAXZ = 100
NSPH = 8
E = N * K

NW = 32
NPAD = 10240
PNW = NPAD // NW
VD = NSPH * HID
NB = 8

BNP = 1024
BNE = 256

_ALPHA = 5.0 / CUTOFF
_MEANS = np.linspace(float(np.exp(-CUTOFF)), 1.0, NRBF)
_BETA = float((2.0 / NRBF * (1.0 - np.exp(-CUTOFF))) ** -2)
_S3 = float(np.sqrt(3.0))
_ISQ = float(1.0 / np.sqrt(DH))


def _silu(x):
    return x * jax.nn.sigmoid(x)


def _build_graph(pos):
    norms = jnp.sum(pos * pos, axis=1)
    idx_list, valid_list = [], []
    B = 1000
    for s in range(0, N, B):
        blk = pos[s:s + B]
        d2 = norms[s:s + B, None] + norms[None, :] - 2.0 * (blk @ pos.T)
        rows = jnp.arange(B)
        d2 = d2.at[rows, rows + s].set(jnp.inf)
        d2 = jnp.where(d2 <= CUTOFF * CUTOFF, d2, jnp.inf)
        neg, idx = jax.lax.top_k(-d2, K)
        idx_list.append(idx)
        valid_list.append(jnp.isfinite(neg))
    nbr = jnp.concatenate(idx_list, axis=0)
    valid = jnp.concatenate(valid_list, axis=0)
    return nbr.reshape(-1), valid.reshape(-1)


G = 10
RB = 125


def _edge_feat_body(rv_ref, mask_ref, cw_ref, rbf_ref, sph_ref):
    x = rv_ref[0, 0]
    y = rv_ref[1, 0]
    z = rv_ref[2, 0]
    d = jnp.sqrt(x * x + y * y + z * z + 1e-12)
    inv = 1.0 / d
    ux, uy, uz = x * inv, y * inv, z * inv
    cw = 0.5 * (jnp.cos(np.pi * jnp.minimum(d, CUTOFF) / CUTOFF) + 1.0)
    cw = jnp.where((mask_ref[0] > 0.5) & (d < CUTOFF), cw, 0.0)
    cw_ref[0] = cw
    ed = jnp.exp(-_ALPHA * d)
    for j in range(NRBF):
        rbf_ref[j, 0] = cw * jnp.exp(-_BETA * (ed - _MEANS[j]) ** 2)
    sph_ref[0, 0] = cw * ux
    sph_ref[1, 0] = cw * uy
    sph_ref[2, 0] = cw * uz
    sph_ref[3, 0] = cw * (_S3 * ux * uy)
    sph_ref[4, 0] = cw * (_S3 * uy * uz)
    sph_ref[5, 0] = cw * (0.5 * (3.0 * uz * uz - 1.0))
    sph_ref[6, 0] = cw * (_S3 * ux * uz)
    sph_ref[7, 0] = cw * (0.5 * _S3 * (ux * ux - uy * uy))


def _edge_features(rvec, maskf):
    rv3 = rvec.T.reshape(3, G, RB, 128)
    m2 = maskf.reshape(G, RB, 128)
    cw, rbf, sph = pl.pallas_call(
        _edge_feat_body,
        grid=(G,),
        in_specs=[
            pl.BlockSpec((3, 1, RB, 128), lambda i: (0, i, 0, 0)),
            pl.BlockSpec((1, RB, 128), lambda i: (i, 0, 0)),
        ],
        out_specs=[
            pl.BlockSpec((1, RB, 128), lambda i: (i, 0, 0)),
            pl.BlockSpec((NRBF, 1, RB, 128), lambda i: (0, i, 0, 0)),
            pl.BlockSpec((NSPH, 1, RB, 128), lambda i: (0, i, 0, 0)),
        ],
        out_shape=[
            jax.ShapeDtypeStruct((G, RB, 128), jnp.float32),
            jax.ShapeDtypeStruct((NRBF, G, RB, 128), jnp.float32),
            jax.ShapeDtypeStruct((NSPH, G, RB, 128), jnp.float32),
        ],
    )(rv3, m2)
    cw = cw.reshape(E)
    rbf = rbf.reshape(NRBF, E).T
    sph = sph.reshape(NSPH, E).T
    return cw, rbf, sph


def _tc_pre_body(x_ref, vec_ref, wq_ref, wk_ref, wv_ref, wvec_ref,
                 lnw_ref, lnb_ref, q_ref, kv_ref, vdot_ref, vec3_ref):
    x = x_ref[...]
    m = jnp.mean(x, axis=1, keepdims=True)
    c = x - m
    var = jnp.mean(c * c, axis=1, keepdims=True)
    h = c / jnp.sqrt(var + 1e-5) * lnw_ref[...] + lnb_ref[...]
    f32 = jnp.float32
    q_ref[...] = jnp.dot(h, wq_ref[...], preferred_element_type=f32)
    kv_ref[:, :HID] = jnp.dot(h, wk_ref[...], preferred_element_type=f32)
    kv_ref[:, HID:] = jnp.dot(h, wv_ref[...], preferred_element_type=f32)
    wvec = wvec_ref[...]
    vdot = jnp.zeros_like(x)
    for s in range(NSPH):
        vs = vec_ref[:, s * HID:(s + 1) * HID]
        vp = jnp.dot(vs, wvec, preferred_element_type=f32)
        vdot = vdot + vp[:, :HID] * vp[:, HID:2 * HID]
        vec3_ref[:, s * HID:(s + 1) * HID] = vp[:, 2 * HID:]
    vdot_ref[...] = vdot


def _tc_pre(x, vec2d, lp):
    grid = (NPAD // BNP,)
    bn = lambda f: pl.BlockSpec((BNP, f), lambda i: (i, 0))
    full = lambda a, b: pl.BlockSpec((a, b), lambda i: (0, 0))
    return pl.pallas_call(
        _tc_pre_body,
        grid=grid,
        in_specs=[bn(HID), bn(VD), full(HID, HID), full(HID, HID),
                  full(HID, HID), full(HID, 3 * HID), full(1, HID),
                  full(1, HID)],
        out_specs=[bn(HID), bn(2 * HID), bn(HID), bn(VD)],
        out_shape=[
            jax.ShapeDtypeStruct((NPAD, HID), jnp.float32),
            jax.ShapeDtypeStruct((NPAD, 2 * HID), jnp.float32),
            jax.ShapeDtypeStruct((NPAD, HID), jnp.float32),
            jax.ShapeDtypeStruct((NPAD, VD), jnp.float32),
        ],
    )(x, vec2d, lp['Wq'], lp['Wk'], lp['Wv'], lp['Wvec'],
      lp['ln_w'].reshape(1, HID), lp['ln_b'].reshape(1, HID))


def _kv_gather_body(kv_hbm, idx_hbm, out_hbm, idx_v, gbuf, sbuf, gsem, ssem):
    wid = lax.axis_index("s") * 2 + lax.axis_index("c")
    base = wid * PNW
    pltpu.sync_copy(idx_hbm.at[pl.ds(base, PNW)], idx_v)

    def issue(b, i):
        pltpu.make_async_copy(kv_hbm.at[idx_v[i]], gbuf.at[b],
                              gsem.at[b]).start()

    for b in range(NB):
        issue(b, b)

    def outer(g, _):
        for b in range(NB):
            i = g * NB + b
            pltpu.make_async_copy(kv_hbm.at[pl.ds(0, K)], gbuf.at[b],
                                  gsem.at[b]).wait()

            @pl.when(g > 0)
            def _():
                pltpu.make_async_copy(sbuf.at[b],
                                      out_hbm.at[pl.ds(base, 1)],
                                      ssem.at[b]).wait()

            def cp(j, _):
                for k in range(K):
                    sbuf[b, 0, pl.ds(k * 2 * HID + j * 16, 16)] = \
                        gbuf[b, k, pl.ds(j * 16, 16)]
                return 0

            lax.fori_loop(0, 2 * HID // 16, cp, 0)
            pltpu.make_async_copy(sbuf.at[b],
                                  out_hbm.at[pl.ds(base + i, 1)],
                                  ssem.at[b]).start()

            @pl.when(g < PNW // NB - 1)
            def _():
                issue(b, i + NB)

        return 0

    lax.fori_loop(0, PNW // NB, outer, 0)
    for b in range(NB):
        pltpu.make_async_copy(sbuf.at[b], out_hbm.at[pl.ds(base, 1)],
                              ssem.at[b]).wait()


@functools.lru_cache(maxsize=None)
def _kv_gather_sc_build():
    return pl.kernel(
        _kv_gather_body,
        mesh=plsc.VectorSubcoreMesh(core_axis_name="c", subcore_axis_name="s"),
        out_type=jax.ShapeDtypeStruct((NPAD, K * 2 * HID), jnp.float32),
        scratch_types=[
            pltpu.VMEM((PNW, K), jnp.int32),
            pltpu.VMEM((NB, K, 2 * HID), jnp.float32),
            pltpu.VMEM((NB, 1, K * 2 * HID), jnp.float32),
            pltpu.SemaphoreType.DMA((NB,)),
            pltpu.SemaphoreType.DMA((NB,)),
        ],
    )


def _kv_gather_sc(kv, idx_pad):
    return _kv_gather_sc_build()(kv, idx_pad)


def _tc_edge_body(q_ref, kvs_ref, rbf_ref, csph_ref, x_ref, vdot_ref,
                  vec_ref, vec3_ref, wdk_ref, wdv_ref, wo_ref,
                  xnew_ref, s1_ref, vbase_ref):
    f32 = jnp.float32
    hsel = (lax.broadcasted_iota(jnp.int32, (HID, NHEADS), 0) // DH ==
            lax.broadcasted_iota(jnp.int32, (HID, NHEADS), 1)).astype(f32)
    hexp = (lax.broadcasted_iota(jnp.int32, (NHEADS, HID), 0) ==
            lax.broadcasted_iota(jnp.int32, (NHEADS, HID), 1) // DH
            ).astype(f32)
    q = q_ref[...]
    wdk = wdk_ref[...]
    wdv = wdv_ref[...]
    xagg = jnp.zeros((BNE, HID), f32)
    t2 = [jnp.zeros((BNE, HID), f32) for _ in range(NSPH)]
    for k in range(K):
        rbf_k = rbf_ref[k]
        dk = _silu(jnp.dot(rbf_k, wdk, preferred_element_type=f32))
        dv = _silu(jnp.dot(rbf_k, wdv, preferred_element_type=f32))
        ks = kvs_ref[:, 2 * HID * k:2 * HID * k + HID]
        vs = kvs_ref[:, 2 * HID * k + HID:2 * HID * (k + 1)]
        prod = q * ks * dk
        lg = jnp.dot(prod, hsel, preferred_element_type=f32) * _ISQ
        a8 = _silu(lg) * csph_ref[k][:, :NSPH]
        a64 = jnp.dot(a8, hexp, preferred_element_type=f32)
        w_ = vs * a64
        xagg = xagg + w_ * dv[:, :HID]
        s1_ref[:, k * HID:(k + 1) * HID] = w_ * dv[:, HID:2 * HID]
        s2_k = w_ * dv[:, 2 * HID:]
        sph_k = csph_ref[k][:, NSPH:]
        for s in range(NSPH):
            t2[s] = t2[s] + sph_k[:, s:s + 1] * s2_k
    o = jnp.dot(xagg, wo_ref[...], preferred_element_type=f32)
    o1 = o[:, :HID]
    xnew_ref[...] = x_ref[...] + o[:, HID:2 * HID] + \
        o[:, 2 * HID:] * vdot_ref[...]
    for s in range(NSPH):
        sl = slice(s * HID, (s + 1) * HID)
        vbase_ref[:, sl] = vec_ref[:, sl] + t2[s] + vec3_ref[:, sl] * o1


def _tc_edge(q, kvsrc, rbf_t, csph, x, vdot, vec2d, vec3, lp):
    grid = (NPAD // BNE,)
    bn = lambda f: pl.BlockSpec((BNE, f), lambda i: (i, 0))
    be = lambda f: pl.BlockSpec((K, BNE, f), lambda i: (0, i, 0))
    full = lambda a, b: pl.BlockSpec((a, b), lambda i: (0, 0))
    return pl.pallas_call(
        _tc_edge_body,
        grid=grid,
        in_specs=[bn(HID), bn(K * 2 * HID), be(NRBF), be(2 * NSPH), bn(HID),
                  bn(HID), bn(VD), bn(VD), full(NRBF, HID),
                  full(NRBF, 3 * HID), full(HID, 3 * HID)],
        out_specs=[bn(HID), bn(K * HID), bn(VD)],
        out_shape=[
            jax.ShapeDtypeStruct((NPAD, HID), jnp.float32),
            jax.ShapeDtypeStruct((NPAD, K * HID), jnp.float32),
            jax.ShapeDtypeStruct((NPAD, VD), jnp.float32),
        ],
    )(q, kvsrc, rbf_t, csph, x, vdot, vec2d,
      vec3, lp['Wdk'], lp['Wdv_perm'], lp['Wo'])


def _vec_agg_body(vec_hbm, s1_hbm, idx_hbm, out_hbm, idx_v, gbuf,
                  s1buf, ostage, gsem, ssem, osem):
    wid = lax.axis_index("s") * 2 + lax.axis_index("c")
    base = wid * PNW
    pltpu.sync_copy(idx_hbm.at[pl.ds(base, PNW)], idx_v)

    def issue(b, i):
        pltpu.make_async_copy(vec_hbm.at[idx_v[i]], gbuf.at[b],
                              gsem.at[b]).start()
        pltpu.make_async_copy(s1_hbm.at[pl.ds(base + i, 1)], s1buf.at[b],
                              ssem.at[b]).start()

    for b in range(NB):
        issue(b, b)

    def compute(b, i):
        pltpu.make_async_copy(vec_hbm.at[pl.ds(0, K)], gbuf.at[b],
                              gsem.at[b]).wait()
        pltpu.make_async_copy(s1_hbm.at[pl.ds(0, 1)], s1buf.at[b],
                              ssem.at[b]).wait()

        def col(c, _):
            s = c // 4
            j = c - s * 4
            off = s * HID + j * 16
            acc = jnp.zeros((16,), jnp.float32)
            for k in range(K):
                acc = acc + (gbuf[b, k, pl.ds(off, 16)]
                             * s1buf[b, 0, pl.ds(k * HID + j * 16, 16)])
            ostage[b, 0, pl.ds(off, 16)] = acc
            return 0

        lax.fori_loop(0, NSPH * 4, col, 0)
        pltpu.make_async_copy(ostage.at[b], out_hbm.at[pl.ds(base + i, 1)],
                              osem.at[b]).start()

    def outer(g, _):
        for b in range(NB):
            i = g * NB + b

            @pl.when(g > 0)
            def _():
                pltpu.make_async_copy(
                    ostage.at[b], out_hbm.at[pl.ds(base, 1)],
                    osem.at[b]).wait()

            compute(b, i)

            @pl.when(g < PNW // NB - 1)
            def _():
                issue(b, i + NB)

        return 0

    lax.fori_loop(0, PNW // NB, outer, 0)
    for b in range(NB):
        pltpu.make_async_copy(ostage.at[b], out_hbm.at[pl.ds(base, 1)],
                              osem.at[b]).wait()


@functools.lru_cache(maxsize=None)
def _vec_agg_sc_build():
    return pl.kernel(
        _vec_agg_body,
        mesh=plsc.VectorSubcoreMesh(core_axis_name="c", subcore_axis_name="s"),
        out_type=jax.ShapeDtypeStruct((NPAD, VD), jnp.float32),
        scratch_types=[
            pltpu.VMEM((PNW, K), jnp.int32),
            pltpu.VMEM((NB, K, VD), jnp.float32),
            pltpu.VMEM((NB, 1, K * HID), jnp.float32),
            pltpu.VMEM((NB, 1, VD), jnp.float32),
            pltpu.SemaphoreType.DMA((NB,)),
            pltpu.SemaphoreType.DMA((NB,)),
            pltpu.SemaphoreType.DMA((NB,)),
        ],
    )


def _vec_agg_sc(vec2d, s1flat, idx_pad):
    return _vec_agg_sc_build()(vec2d, s1flat, idx_pad)


def _layer_full(x, vec2d, lp, idx_pad, rbf_t, csph):
    q, kv, vdot, vec3 = _tc_pre(x, vec2d, lp)
    kvsrc = _kv_gather_sc(kv, idx_pad)
    x_new, s1, vbase = _tc_edge(q, kvsrc, rbf_t, csph, x, vdot, vec2d,
                                vec3, lp)
    vec_new = vbase + _vec_agg_sc(vec2d, s1, idx_pad)
    return x_new, vec_new


def _dv_perm():
    p = np.zeros(3 * HID, np.int32)
    for j in range(HID):
        h, d = divmod(j, DH)
        p[j] = 3 * DH * h + d
        p[HID + j] = 3 * DH * h + DH + d
        p[2 * HID + j] = 3 * DH * h + 2 * DH + d
    return p


_DVPERM = _dv_perm()


def kernel(z, pos, batch, params):
    src, mask = _build_graph(pos)
    dst = jnp.repeat(jnp.arange(N), K)
    rvec = pos[src] - pos[dst]
    cw, rbf, sph = _edge_features(rvec, mask.astype(jnp.float32))

    pad_nk = lambda a: jnp.pad(a.reshape(N, K, -1),
                               ((0, NPAD - N), (0, 0), (0, 0)))
    rbf_t = jnp.transpose(pad_nk(rbf), (1, 0, 2))
    cw_t = jnp.transpose(pad_nk(cw), (1, 0, 2))
    sph_t = jnp.transpose(pad_nk(sph), (1, 0, 2))
    csph = jnp.concatenate(
        [jnp.broadcast_to(cw_t, (K, NPAD, NSPH)), sph_t], axis=2)
    idx_pad = jnp.pad(src.reshape(N, K).astype(jnp.int32),
                      ((0, NPAD - N), (0, 0)))

    x = jnp.pad(params['embed'][z], ((0, NPAD - N), (0, 0)))
    vec2d = jnp.zeros((NPAD, VD), jnp.float32)
    lkeys = ['Wq', 'Wk', 'Wv', 'Wdk', 'Wvec', 'Wo', 'ln_w', 'ln_b']
    for i in range(NLAYERS):
        lp = {kk: params[kk][i] for kk in lkeys}
        lp['Wdv_perm'] = params['Wdv'][i][:, _DVPERM]
        x, vec2d = _layer_full(x, vec2d, lp, idx_pad, rbf_t, csph)

    h = _silu(x[:N] @ params['Wout1'] + params['bout1'])
    atom_energy = h @ params['Wout2'] + params['bout2']
    return jax.ops.segment_sum(atom_energy, batch, num_segments=NG)


# speedup vs baseline: 18.5231x; 2.0314x over previous
import functools

import numpy as np
import jax
import jax.numpy as jnp
from jax import lax
from jax.experimental import pallas as pl
from jax.experimental.pallas import tpu as pltpu
from jax.experimental.pallas import tpu_sc as plsc

N = 10000
NG = 512
HID = 64
NRBF = 32
NLAYERS = 6
NHEADS = 8
DH = HID // NHEADS
K = 16
CUTOFF = 5.0
MAXZ = 100
NSPH = 8
E = N * K

NW = 32
NPAD = 10240
PNW = NPAD // NW
VD = NSPH * HID
NB = 8

BNP = 1024
BNE = 256

_ALPHA = 5.0 / CUTOFF
_MEANS = np.linspace(float(np.exp(-CUTOFF)), 1.0, NRBF)
_BETA = float((2.0 / NRBF * (1.0 - np.exp(-CUTOFF))) ** -2)
_S3 = float(np.sqrt(3.0))
_ISQ = float(1.0 / np.sqrt(DH))


def _silu(x):
    return x * jax.nn.sigmoid(x)


BGR = 80
_C2 = CUTOFF * CUTOFF
_FINF = float(np.float32(3.0e38))


def _graph_body(d2_ref, idx_ref, val_ref):
    i = pl.program_id(0)
    f32 = jnp.float32
    cols = lax.broadcasted_iota(jnp.int32, (BGR, N), 1)
    rows = lax.broadcasted_iota(jnp.int32, (BGR, N), 0) + i * BGR
    d2 = jnp.where((cols == rows) | (d2_ref[...] > _C2), _FINF, d2_ref[...])
    for k in range(K):
        m = jnp.min(d2, axis=1, keepdims=True)
        am = jnp.min(jnp.where(d2 == m, cols, N), axis=1, keepdims=True)
        idx_ref[:, k:k + 1] = am
        val_ref[:, k:k + 1] = (m <= _C2).astype(f32)
        d2 = jnp.where(cols == am, _FINF, d2)


def _build_graph(pos):
    norms = jnp.sum(pos * pos, axis=1)
    B = 1000
    d2 = jnp.concatenate(
        [norms[s:s + B, None] + norms[None, :] - 2.0 * (pos[s:s + B] @ pos.T)
         for s in range(0, N, B)], axis=0)
    idx, val = pl.pallas_call(
        _graph_body,
        grid=(N // BGR,),
        in_specs=[pl.BlockSpec((BGR, N), lambda i: (i, 0))],
        out_specs=[
            pl.BlockSpec((BGR, K), lambda i: (i, 0)),
            pl.BlockSpec((BGR, K), lambda i: (i, 0)),
        ],
        out_shape=[
            jax.ShapeDtypeStruct((N, K), jnp.int32),
            jax.ShapeDtypeStruct((N, K), jnp.float32),
        ],
    )(d2)
    return idx.reshape(-1), val.reshape(-1) > 0.5


G = 10
RB = 125


def _edge_feat_body(rv_ref, mask_ref, cw_ref, rbf_ref, sph_ref):
    x = rv_ref[0, 0]
    y = rv_ref[1, 0]
    z = rv_ref[2, 0]
    d = jnp.sqrt(x * x + y * y + z * z + 1e-12)
    inv = 1.0 / d
    ux, uy, uz = x * inv, y * inv, z * inv
    cw = 0.5 * (jnp.cos(np.pi * jnp.minimum(d, CUTOFF) / CUTOFF) + 1.0)
    cw = jnp.where((mask_ref[0] > 0.5) & (d < CUTOFF), cw, 0.0)
    cw_ref[0] = cw
    ed = jnp.exp(-_ALPHA * d)
    for j in range(NRBF):
        rbf_ref[j, 0] = cw * jnp.exp(-_BETA * (ed - _MEANS[j]) ** 2)
    sph_ref[0, 0] = cw * ux
    sph_ref[1, 0] = cw * uy
    sph_ref[2, 0] = cw * uz
    sph_ref[3, 0] = cw * (_S3 * ux * uy)
    sph_ref[4, 0] = cw * (_S3 * uy * uz)
    sph_ref[5, 0] = cw * (0.5 * (3.0 * uz * uz - 1.0))
    sph_ref[6, 0] = cw * (_S3 * ux * uz)
    sph_ref[7, 0] = cw * (0.5 * _S3 * (ux * ux - uy * uy))


def _edge_features(rvec, maskf):
    rv3 = rvec.T.reshape(3, G, RB, 128)
    m2 = maskf.reshape(G, RB, 128)
    cw, rbf, sph = pl.pallas_call(
        _edge_feat_body,
        grid=(G,),
        in_specs=[
            pl.BlockSpec((3, 1, RB, 128), lambda i: (0, i, 0, 0)),
            pl.BlockSpec((1, RB, 128), lambda i: (i, 0, 0)),
        ],
        out_specs=[
            pl.BlockSpec((1, RB, 128), lambda i: (i, 0, 0)),
            pl.BlockSpec((NRBF, 1, RB, 128), lambda i: (0, i, 0, 0)),
            pl.BlockSpec((NSPH, 1, RB, 128), lambda i: (0, i, 0, 0)),
        ],
        out_shape=[
            jax.ShapeDtypeStruct((G, RB, 128), jnp.float32),
            jax.ShapeDtypeStruct((NRBF, G, RB, 128), jnp.float32),
            jax.ShapeDtypeStruct((NSPH, G, RB, 128), jnp.float32),
        ],
    )(rv3, m2)
    cw = cw.reshape(E)
    rbf = rbf.reshape(NRBF, E).T
    sph = sph.reshape(NSPH, E).T
    return cw, rbf, sph


def _tc_pre_body(x_ref, vec_ref, wq_ref, wk_ref, wv_ref, wvec_ref,
                 lnw_ref, lnb_ref, q_ref, kv_ref, vdot_ref, vec3_ref):
    x = x_ref[...]
    m = jnp.mean(x, axis=1, keepdims=True)
    c = x - m
    var = jnp.mean(c * c, axis=1, keepdims=True)
    h = c / jnp.sqrt(var + 1e-5) * lnw_ref[...] + lnb_ref[...]
    f32 = jnp.float32
    q_ref[...] = jnp.dot(h, wq_ref[...], preferred_element_type=f32)
    kv_ref[:, :HID] = jnp.dot(h, wk_ref[...], preferred_element_type=f32)
    kv_ref[:, HID:] = jnp.dot(h, wv_ref[...], preferred_element_type=f32)
    wvec = wvec_ref[...]
    vdot = jnp.zeros_like(x)
    for s in range(NSPH):
        vs = vec_ref[:, s * HID:(s + 1) * HID]
        vp = jnp.dot(vs, wvec, preferred_element_type=f32)
        vdot = vdot + vp[:, :HID] * vp[:, HID:2 * HID]
        vec3_ref[:, s * HID:(s + 1) * HID] = vp[:, 2 * HID:]
    vdot_ref[...] = vdot


def _tc_pre(x, vec2d, lp):
    grid = (NPAD // BNP,)
    bn = lambda f: pl.BlockSpec((BNP, f), lambda i: (i, 0))
    full = lambda a, b: pl.BlockSpec((a, b), lambda i: (0, 0))
    return pl.pallas_call(
        _tc_pre_body,
        grid=grid,
        in_specs=[bn(HID), bn(VD), full(HID, HID), full(HID, HID),
                  full(HID, HID), full(HID, 3 * HID), full(1, HID),
                  full(1, HID)],
        out_specs=[bn(HID), bn(2 * HID), bn(HID), bn(VD)],
        out_shape=[
            jax.ShapeDtypeStruct((NPAD, HID), jnp.float32),
            jax.ShapeDtypeStruct((NPAD, 2 * HID), jnp.float32),
            jax.ShapeDtypeStruct((NPAD, HID), jnp.float32),
            jax.ShapeDtypeStruct((NPAD, VD), jnp.float32),
        ],
    )(x, vec2d, lp['Wq'], lp['Wk'], lp['Wv'], lp['Wvec'],
      lp['ln_w'].reshape(1, HID), lp['ln_b'].reshape(1, HID))


def _kv_gather_body(kv_hbm, idx_hbm, out_hbm, idx_v, gbuf, sbuf, gsem, ssem):
    wid = lax.axis_index("s") * 2 + lax.axis_index("c")
    base = wid * PNW
    pltpu.sync_copy(idx_hbm.at[pl.ds(base, PNW)], idx_v)

    def issue(b, i):
        pltpu.make_async_copy(kv_hbm.at[idx_v[i]], gbuf.at[b],
                              gsem.at[b]).start()

    for b in range(NB):
        issue(b, b)

    def outer(g, _):
        for b in range(NB):
            i = g * NB + b
            pltpu.make_async_copy(kv_hbm.at[pl.ds(0, K)], gbuf.at[b],
                                  gsem.at[b]).wait()

            @pl.when(g > 0)
            def _():
                pltpu.make_async_copy(sbuf.at[b],
                                      out_hbm.at[pl.ds(base, 1)],
                                      ssem.at[b]).wait()

            def cp(j, _):
                for k in range(K):
                    sbuf[b, 0, pl.ds(k * 2 * HID + j * 16, 16)] = \
                        gbuf[b, k, pl.ds(j * 16, 16)]
                return 0

            lax.fori_loop(0, 2 * HID // 16, cp, 0)
            pltpu.make_async_copy(sbuf.at[b],
                                  out_hbm.at[pl.ds(base + i, 1)],
                                  ssem.at[b]).start()

            @pl.when(g < PNW // NB - 1)
            def _():
                issue(b, i + NB)

        return 0

    lax.fori_loop(0, PNW // NB, outer, 0)
    for b in range(NB):
        pltpu.make_async_copy(sbuf.at[b], out_hbm.at[pl.ds(base, 1)],
                              ssem.at[b]).wait()


@functools.lru_cache(maxsize=None)
def _kv_gather_sc_build():
    return pl.kernel(
        _kv_gather_body,
        mesh=plsc.VectorSubcoreMesh(core_axis_name="c", subcore_axis_name="s"),
        out_type=jax.ShapeDtypeStruct((NPAD, K * 2 * HID), jnp.float32),
        scratch_types=[
            pltpu.VMEM((PNW, K), jnp.int32),
            pltpu.VMEM((NB, K, 2 * HID), jnp.float32),
            pltpu.VMEM((NB, 1, K * 2 * HID), jnp.float32),
            pltpu.SemaphoreType.DMA((NB,)),
            pltpu.SemaphoreType.DMA((NB,)),
        ],
    )


def _kv_gather_sc(kv, idx_pad):
    return _kv_gather_sc_build()(kv, idx_pad)


def _tc_edge_body(q_ref, kvs_ref, rbf_ref, csph_ref, x_ref, vdot_ref,
                  vec_ref, vec3_ref, wdk_ref, wdv_ref, wo_ref,
                  xnew_ref, s1_ref, vbase_ref):
    f32 = jnp.float32
    hsel = (lax.broadcasted_iota(jnp.int32, (HID, NHEADS), 0) // DH ==
            lax.broadcasted_iota(jnp.int32, (HID, NHEADS), 1)).astype(f32)
    hexp = (lax.broadcasted_iota(jnp.int32, (NHEADS, HID), 0) ==
            lax.broadcasted_iota(jnp.int32, (NHEADS, HID), 1) // DH
            ).astype(f32)
    q = q_ref[...]
    wdk = wdk_ref[...]
    wdv = wdv_ref[...]
    xagg = jnp.zeros((BNE, HID), f32)
    t2 = [jnp.zeros((BNE, HID), f32) for _ in range(NSPH)]
    for k in range(K):
        rbf_k = rbf_ref[k]
        dk = _silu(jnp.dot(rbf_k, wdk, preferred_element_type=f32))
        dv = _silu(jnp.dot(rbf_k, wdv, preferred_element_type=f32))
        ks = kvs_ref[:, 2 * HID * k:2 * HID * k + HID]
        vs = kvs_ref[:, 2 * HID * k + HID:2 * HID * (k + 1)]
        prod = q * ks * dk
        lg = jnp.dot(prod, hsel, preferred_element_type=f32) * _ISQ
        a8 = _silu(lg) * csph_ref[k][:, :NSPH]
        a64 = jnp.dot(a8, hexp, preferred_element_type=f32)
        w_ = vs * a64
        xagg = xagg + w_ * dv[:, :HID]
        s1_ref[:, k * HID:(k + 1) * HID] = w_ * dv[:, HID:2 * HID]
        s2_k = w_ * dv[:, 2 * HID:]
        sph_k = csph_ref[k][:, NSPH:]
        for s in range(NSPH):
            t2[s] = t2[s] + sph_k[:, s:s + 1] * s2_k
    o = jnp.dot(xagg, wo_ref[...], preferred_element_type=f32)
    o1 = o[:, :HID]
    xnew_ref[...] = x_ref[...] + o[:, HID:2 * HID] + \
        o[:, 2 * HID:] * vdot_ref[...]
    for s in range(NSPH):
        sl = slice(s * HID, (s + 1) * HID)
        vbase_ref[:, sl] = vec_ref[:, sl] + t2[s] + vec3_ref[:, sl] * o1


def _tc_edge(q, kvsrc, rbf_t, csph, x, vdot, vec2d, vec3, lp):
    grid = (NPAD // BNE,)
    bn = lambda f: pl.BlockSpec((BNE, f), lambda i: (i, 0))
    be = lambda f: pl.BlockSpec((K, BNE, f), lambda i: (0, i, 0))
    full = lambda a, b: pl.BlockSpec((a, b), lambda i: (0, 0))
    return pl.pallas_call(
        _tc_edge_body,
        grid=grid,
        in_specs=[bn(HID), bn(K * 2 * HID), be(NRBF), be(2 * NSPH), bn(HID),
                  bn(HID), bn(VD), bn(VD), full(NRBF, HID),
                  full(NRBF, 3 * HID), full(HID, 3 * HID)],
        out_specs=[bn(HID), bn(K * HID), bn(VD)],
        out_shape=[
            jax.ShapeDtypeStruct((NPAD, HID), jnp.float32),
            jax.ShapeDtypeStruct((NPAD, K * HID), jnp.float32),
            jax.ShapeDtypeStruct((NPAD, VD), jnp.float32),
        ],
    )(q, kvsrc, rbf_t, csph, x, vdot, vec2d,
      vec3, lp['Wdk'], lp['Wdv_perm'], lp['Wo'])


def _vec_agg_body(vec_hbm, s1_hbm, idx_hbm, out_hbm, idx_v, gbuf,
                  s1buf, ostage, gsem, ssem, osem):
    wid = lax.axis_index("s") * 2 + lax.axis_index("c")
    base = wid * PNW
    pltpu.sync_copy(idx_hbm.at[pl.ds(base, PNW)], idx_v)

    def issue(b, i):
        pltpu.make_async_copy(vec_hbm.at[idx_v[i]], gbuf.at[b],
                              gsem.at[b]).start()
        pltpu.make_async_copy(s1_hbm.at[pl.ds(base + i, 1)], s1buf.at[b],
                              ssem.at[b]).start()

    for b in range(NB):
        issue(b, b)

    def compute(b, i):
        pltpu.make_async_copy(vec_hbm.at[pl.ds(0, K)], gbuf.at[b],
                              gsem.at[b]).wait()
        pltpu.make_async_copy(s1_hbm.at[pl.ds(0, 1)], s1buf.at[b],
                              ssem.at[b]).wait()

        def col(c, _):
            s = c // 4
            j = c - s * 4
            off = s * HID + j * 16
            acc = jnp.zeros((16,), jnp.float32)
            for k in range(K):
                acc = acc + (gbuf[b, k, pl.ds(off, 16)]
                             * s1buf[b, 0, pl.ds(k * HID + j * 16, 16)])
            ostage[b, 0, pl.ds(off, 16)] = acc
            return 0

        lax.fori_loop(0, NSPH * 4, col, 0)
        pltpu.make_async_copy(ostage.at[b], out_hbm.at[pl.ds(base + i, 1)],
                              osem.at[b]).start()

    def outer(g, _):
        for b in range(NB):
            i = g * NB + b

            @pl.when(g > 0)
            def _():
                pltpu.make_async_copy(
                    ostage.at[b], out_hbm.at[pl.ds(base, 1)],
                    osem.at[b]).wait()

            compute(b, i)

            @pl.when(g < PNW // NB - 1)
            def _():
                issue(b, i + NB)

        return 0

    lax.fori_loop(0, PNW // NB, outer, 0)
    for b in range(NB):
        pltpu.make_async_copy(ostage.at[b], out_hbm.at[pl.ds(base, 1)],
                              osem.at[b]).wait()


@functools.lru_cache(maxsize=None)
def _vec_agg_sc_build():
    return pl.kernel(
        _vec_agg_body,
        mesh=plsc.VectorSubcoreMesh(core_axis_name="c", subcore_axis_name="s"),
        out_type=jax.ShapeDtypeStruct((NPAD, VD), jnp.float32),
        scratch_types=[
            pltpu.VMEM((PNW, K), jnp.int32),
            pltpu.VMEM((NB, K, VD), jnp.float32),
            pltpu.VMEM((NB, 1, K * HID), jnp.float32),
            pltpu.VMEM((NB, 1, VD), jnp.float32),
            pltpu.SemaphoreType.DMA((NB,)),
            pltpu.SemaphoreType.DMA((NB,)),
            pltpu.SemaphoreType.DMA((NB,)),
        ],
    )


def _vec_agg_sc(vec2d, s1flat, idx_pad):
    return _vec_agg_sc_build()(vec2d, s1flat, idx_pad)


def _layer_full(x, vec2d, lp, idx_pad, rbf_t, csph):
    q, kv, vdot, vec3 = _tc_pre(x, vec2d, lp)
    kvsrc = _kv_gather_sc(kv, idx_pad)
    x_new, s1, vbase = _tc_edge(q, kvsrc, rbf_t, csph, x, vdot, vec2d,
                                vec3, lp)
    vec_new = vbase + _vec_agg_sc(vec2d, s1, idx_pad)
    return x_new, vec_new


def _dv_perm():
    p = np.zeros(3 * HID, np.int32)
    for j in range(HID):
        h, d = divmod(j, DH)
        p[j] = 3 * DH * h + d
        p[HID + j] = 3 * DH * h + DH + d
        p[2 * HID + j] = 3 * DH * h + 2 * DH + d
    return p


_DVPERM = _dv_perm()


def kernel(z, pos, batch, params):
    src, mask = _build_graph(pos)
    dst = jnp.repeat(jnp.arange(N), K)
    rvec = pos[src] - pos[dst]
    cw, rbf, sph = _edge_features(rvec, mask.astype(jnp.float32))

    pad_nk = lambda a: jnp.pad(a.reshape(N, K, -1),
                               ((0, NPAD - N), (0, 0), (0, 0)))
    rbf_t = jnp.transpose(pad_nk(rbf), (1, 0, 2))
    cw_t = jnp.transpose(pad_nk(cw), (1, 0, 2))
    sph_t = jnp.transpose(pad_nk(sph), (1, 0, 2))
    csph = jnp.concatenate(
        [jnp.broadcast_to(cw_t, (K, NPAD, NSPH)), sph_t], axis=2)
    idx_pad = jnp.pad(src.reshape(N, K).astype(jnp.int32),
                      ((0, NPAD - N), (0, 0)))

    x = jnp.pad(params['embed'][z], ((0, NPAD - N), (0, 0)))
    vec2d = jnp.zeros((NPAD, VD), jnp.float32)
    lkeys = ['Wq', 'Wk', 'Wv', 'Wdk', 'Wvec', 'Wo', 'ln_w', 'ln_b']
    for i in range(NLAYERS):
        lp = {kk: params[kk][i] for kk in lkeys}
        lp['Wdv_perm'] = params['Wdv'][i][:, _DVPERM]
        x, vec2d = _layer_full(x, vec2d, lp, idx_pad, rbf_t, csph)

    h = _silu(x[:N] @ params['Wout1'] + params['bout1'])
    atom_energy = h @ params['Wout2'] + params['bout2']
    return jax.ops.segment_sum(atom_energy, batch, num_segments=NG)
